# baseline (speedup 1.0000x reference)
import jax
import jax.numpy as jnp
from jax.experimental import pallas as pl
from jax.experimental.pallas import tpu as pltpu

N = 10000
BMAX = 400
RAMP = [80, 160, 240, 320]
SIZES = RAMP + [BMAX] * ((N - 2 * sum(RAMP)) // BMAX) + RAMP[::-1]
assert sum(SIZES) == N
OFFS = [sum(SIZES[:j]) for j in range(len(SIZES))]
NBUF = 3


def _gcn_kernel(x_hbm, adj_hbm, w_ref, b_ref, out_ref, xv_ref, sup_ref,
                buf_ref, sem, xsem):
    def start_copy(step):
        sz, off = SIZES[step], OFFS[step]
        slot = step % NBUF
        pltpu.make_async_copy(
            adj_hbm.at[pl.ds(off, sz), :],
            buf_ref.at[slot, pl.ds(0, sz), :],
            sem.at[slot],
        ).start()

    def wait_copy(step):
        sz, off = SIZES[step], OFFS[step]
        slot = step % NBUF
        pltpu.make_async_copy(
            adj_hbm.at[pl.ds(off, sz), :],
            buf_ref.at[slot, pl.ds(0, sz), :],
            sem.at[slot],
        ).wait()

    xcopy = pltpu.make_async_copy(x_hbm, xv_ref, xsem)
    xcopy.start()
    for j in range(NBUF):
        start_copy(j)
    xcopy.wait()
    sup_ref[:, :] = jnp.dot(
        xv_ref[:, :], w_ref[:, :], preferred_element_type=jnp.float32
    )

    for step, (sz, off) in enumerate(zip(SIZES, OFFS)):
        wait_copy(step)
        h = jnp.dot(
            buf_ref[step % NBUF, 0:sz, :],
            sup_ref[:, :],
            preferred_element_type=jnp.float32,
        )
        h = jax.nn.relu(h + b_ref[:, :])
        m = jnp.max(h, axis=1, keepdims=True)
        lse = jnp.log(jnp.sum(jnp.exp(h - m), axis=1, keepdims=True)) + m
        out_ref[pl.ds(off, sz), :] = h - lse
        if step + NBUF < len(SIZES):
            start_copy(step + NBUF)


@jax.jit
def _run(x, adj, W, b):
    nhid = W.shape[1]
    nfeat = x.shape[1]
    return pl.pallas_call(
        _gcn_kernel,
        in_specs=[
            pl.BlockSpec(memory_space=pl.ANY),
            pl.BlockSpec(memory_space=pl.ANY),
            pl.BlockSpec(memory_space=pltpu.VMEM),
            pl.BlockSpec(memory_space=pltpu.VMEM),
        ],
        out_specs=pl.BlockSpec(memory_space=pltpu.VMEM),
        out_shape=jax.ShapeDtypeStruct((N, nhid), jnp.float32),
        scratch_shapes=[
            pltpu.VMEM((N, nfeat), jnp.float32),
            pltpu.VMEM((N, nhid), jnp.float32),
            pltpu.VMEM((NBUF, BMAX, N), jnp.float32),
            pltpu.SemaphoreType.DMA((NBUF,)),
            pltpu.SemaphoreType.DMA,
        ],
        compiler_params=pltpu.CompilerParams(
            vmem_limit_bytes=100 * 1024 * 1024,
        ),
    )(x, adj, W, b)


def kernel(x, adj, W, b):
    return _run(x, adj, W, b.reshape(1, -1))


# speedup vs baseline: 1.0065x; 1.0065x over previous
import jax
import jax.numpy as jnp
from jax.experimental import pallas as pl
from jax.experimental.pallas import tpu as pltpu

N = 10000
BMAX = 240
SIZES = [80, 160] + [BMAX] * 40 + [160]
assert sum(SIZES) == N
OFFS = [sum(SIZES[:j]) for j in range(len(SIZES))]
NBUF = 4


def _gcn_kernel(x_hbm, adj_hbm, w_ref, b_ref, out_ref, xv_ref, sup_ref,
                buf_ref, sem, xsem):
    def copy(step):
        sz, off = SIZES[step], OFFS[step]
        return pltpu.make_async_copy(
            adj_hbm.at[pl.ds(off, sz), :],
            buf_ref.at[step % NBUF, pl.ds(0, sz), :],
            sem.at[step % NBUF],
        )

    xcopy = pltpu.make_async_copy(x_hbm, xv_ref, xsem)
    xcopy.start()
    for j in range(NBUF - 1):
        copy(j).start()
    xcopy.wait()
    sup_ref[:, :] = jnp.dot(
        xv_ref[:, :], w_ref[:, :], preferred_element_type=jnp.float32
    )

    for step, (sz, off) in enumerate(zip(SIZES, OFFS)):
        copy(step).wait()
        if step + NBUF - 1 < len(SIZES):
            copy(step + NBUF - 1).start()
        h = jnp.dot(
            buf_ref[step % NBUF, 0:sz, :],
            sup_ref[:, :],
            preferred_element_type=jnp.float32,
        )
        h = jax.nn.relu(h + b_ref[:, :])
        m = jnp.max(h, axis=1, keepdims=True)
        lse = jnp.log(jnp.sum(jnp.exp(h - m), axis=1, keepdims=True)) + m
        out_ref[pl.ds(off, sz), :] = h - lse


@jax.jit
def _run(x, adj, W, b):
    nhid = W.shape[1]
    nfeat = x.shape[1]
    return pl.pallas_call(
        _gcn_kernel,
        in_specs=[
            pl.BlockSpec(memory_space=pl.ANY),
            pl.BlockSpec(memory_space=pl.ANY),
            pl.BlockSpec(memory_space=pltpu.VMEM),
            pl.BlockSpec(memory_space=pltpu.VMEM),
        ],
        out_specs=pl.BlockSpec(memory_space=pltpu.VMEM),
        out_shape=jax.ShapeDtypeStruct((N, nhid), jnp.float32),
        scratch_shapes=[
            pltpu.VMEM((N, nfeat), jnp.float32),
            pltpu.VMEM((N, nhid), jnp.float32),
            pltpu.VMEM((NBUF, BMAX, N), jnp.float32),
            pltpu.SemaphoreType.DMA((NBUF,)),
            pltpu.SemaphoreType.DMA,
        ],
        compiler_params=pltpu.CompilerParams(
            vmem_limit_bytes=100 * 1024 * 1024,
        ),
    )(x, adj, W, b)


def kernel(x, adj, W, b):
    return _run(x, adj, W, b.reshape(1, -1))
